# baseline (speedup 1.0000x reference)
import functools

import jax
import jax.numpy as jnp
from jax.experimental import pallas as pl
from jax.experimental.pallas import tpu as pltpu

_H = 256
_G4 = 4 * _H
_A_R0, _A_R1, _A_P0, _A_P1 = 0, 2, 32, 96
_LAG = 16


def _push_hh(whh_scr):
    pltpu.matmul_push_rhs(whh_scr[:, 0 * _H:1 * _H],
                          staging_register=0, mxu_index=0)
    pltpu.matmul_push_rhs(whh_scr[:, 2 * _H:3 * _H],
                          staging_register=0, mxu_index=1)
    pltpu.matmul_push_rhs(whh_scr[:, 1 * _H:2 * _H],
                          staging_register=1, mxu_index=0)
    pltpu.matmul_push_rhs(whh_scr[:, 3 * _H:4 * _H],
                          staging_register=1, mxu_index=1)


def _row_copy(tok_ref, emb_ref, x_scr, sem, slot, base, r):
    tok = tok_ref[base + r]
    return pltpu.make_async_copy(
        emb_ref.at[tok], x_scr.at[slot, r], sem)


def _fused_kernel(tok_ref, emb_ref, w_ref, b_ref, h0_ref, c0_ref,
                  out_ref, c_out_ref, h_out_ref,
                  h_scr, c_scr, whh_scr, g_scr, x_scr, sem, *, block_t):
    blk = pl.program_id(0)
    nb = pl.num_programs(0)
    slot = jax.lax.rem(blk, 2)
    nslot = jax.lax.rem(blk + 1, 2)

    @pl.when(blk == 0)
    def _():
        h_scr[...] = jnp.broadcast_to(h0_ref[...] * 2.0, (8, _H))
        c_scr[...] = jnp.broadcast_to(c0_ref[...], (8, _H))
        whh = w_ref[_H:, :]
        whh_scr[:, :2 * _H] = whh[:, :2 * _H] * 0.25
        whh_scr[:, 2 * _H:3 * _H] = whh[:, 2 * _H:3 * _H] * 0.5
        whh_scr[:, 3 * _H:] = whh[:, 3 * _H:] * 0.25
        for r in range(block_t):
            _row_copy(tok_ref, emb_ref, x_scr, sem, 0, 0, r).start()
        for r in range(block_t):
            _row_copy(tok_ref, emb_ref, x_scr, sem, 0, 0, r).wait()

    xb = x_scr[slot]
    pltpu.matmul_push_rhs(w_ref[:_H, 0 * _H:1 * _H],
                          staging_register=0, mxu_index=0)
    pltpu.matmul_push_rhs(w_ref[:_H, 2 * _H:3 * _H],
                          staging_register=0, mxu_index=1)
    pltpu.matmul_acc_lhs(_A_P0, xb, mxu_index=0, load_staged_rhs=0)
    pltpu.matmul_acc_lhs(_A_P0, xb, mxu_index=1, load_staged_rhs=0)
    pltpu.matmul_push_rhs(w_ref[:_H, 1 * _H:2 * _H],
                          staging_register=0, mxu_index=0)
    pltpu.matmul_push_rhs(w_ref[:_H, 3 * _H:4 * _H],
                          staging_register=0, mxu_index=1)
    pltpu.matmul_acc_lhs(_A_P1, xb, mxu_index=0, load_staged_rhs=0)
    pltpu.matmul_acc_lhs(_A_P1, xb, mxu_index=1, load_staged_rhs=0)
    _push_hh(whh_scr)
    gi = pltpu.matmul_pop(_A_P0, (block_t, _H), jnp.float32, mxu_index=0)
    gg = pltpu.matmul_pop(_A_P0, (block_t, _H), jnp.float32, mxu_index=1)
    gf = pltpu.matmul_pop(_A_P1, (block_t, _H), jnp.float32, mxu_index=0)
    go = pltpu.matmul_pop(_A_P1, (block_t, _H), jnp.float32, mxu_index=1)
    b = b_ref[...]
    g_scr[:, 0 * _H:1 * _H] = (gi + b[:, 0 * _H:1 * _H]) * 0.5
    g_scr[:, 1 * _H:2 * _H] = (gf + b[:, 1 * _H:2 * _H]) * 0.5
    g_scr[:, 2 * _H:3 * _H] = gg + b[:, 2 * _H:3 * _H]
    g_scr[:, 3 * _H:4 * _H] = (go + b[:, 3 * _H:4 * _H]) * 0.5

    nbase = (blk + 1) * block_t
    not_last = blk + 1 < nb

    h = h_scr[...]
    c = c_scr[...]
    for j in range(block_t):
        pltpu.matmul_acc_lhs(_A_R0, h, mxu_index=0, load_staged_rhs=0)
        pltpu.matmul_acc_lhs(_A_R0, h, mxu_index=1, load_staged_rhs=0)
        pltpu.matmul_acc_lhs(_A_R1, h, mxu_index=0, load_staged_rhs=1)
        pltpu.matmul_acc_lhs(_A_R1, h, mxu_index=1, load_staged_rhs=1)
        if j + 1 < block_t:
            _push_hh(whh_scr)

        @pl.when(not_last)
        def _():
            _row_copy(tok_ref, emb_ref, x_scr, sem, nslot, nbase, j).start()
            if j >= _LAG:
                _row_copy(tok_ref, emb_ref, x_scr, sem,
                          nslot, nbase, j - _LAG).wait()

        v0 = pltpu.matmul_pop(_A_R0, (16, _H), jnp.float32, mxu_index=0)
        v1 = pltpu.matmul_pop(_A_R0, (16, _H), jnp.float32, mxu_index=1)
        ti = jnp.tanh(v0[0:8] + g_scr[j:j + 1, 0 * _H:1 * _H])
        tf = jnp.tanh(v0[8:16] + g_scr[j:j + 1, 1 * _H:2 * _H])
        g_g = jnp.tanh(v1[0:8] + g_scr[j:j + 1, 2 * _H:3 * _H])
        to = jnp.tanh(v1[8:16] + g_scr[j:j + 1, 3 * _H:4 * _H])
        c = c * (0.5 * tf + 0.5) + g_g * (0.5 * ti + 0.5)
        h = (to + 1.0) * jnp.tanh(c)
        out_ref[j:j + 1, :] = 0.5 * h[0:1, :]
    h_scr[...] = h
    c_scr[...] = c

    @pl.when(not_last)
    def _():
        for r in range(block_t - _LAG, block_t):
            _row_copy(tok_ref, emb_ref, x_scr, sem, nslot, nbase, r).wait()

    @pl.when(blk == nb - 1)
    def _():
        c_out_ref[...] = c[0:1, :]
        h_out_ref[...] = 0.5 * h[0:1, :]


def _pick_block(n, candidates):
    for c in candidates:
        if n % c == 0:
            return c
    return 1


def kernel(emb, w, b, token_ids, h0, c0):
    S = token_ids.shape[0]
    H = _H

    bt = _pick_block(S, (256, 128, 64, 32))
    tok = token_ids.astype(jnp.int32)
    s_pad = S
    if S % bt:
        s_pad = (S // bt + 1) * bt
        tok = jnp.zeros((s_pad,), jnp.int32).at[:S].set(tok)

    h0_2d = h0.reshape(1, H)
    c0_2d = c0.reshape(1, H)
    grid_spec = pltpu.PrefetchScalarGridSpec(
        num_scalar_prefetch=1,
        grid=(s_pad // bt,),
        in_specs=[
            pl.BlockSpec(memory_space=pltpu.MemorySpace.HBM),
            pl.BlockSpec((2 * H, _G4), lambda i, tok_ref: (0, 0)),
            pl.BlockSpec((1, _G4), lambda i, tok_ref: (0, 0)),
            pl.BlockSpec((1, H), lambda i, tok_ref: (0, 0)),
            pl.BlockSpec((1, H), lambda i, tok_ref: (0, 0)),
        ],
        out_specs=[
            pl.BlockSpec((bt, H), lambda i, tok_ref: (i, 0)),
            pl.BlockSpec((1, H), lambda i, tok_ref: (0, 0)),
            pl.BlockSpec((1, H), lambda i, tok_ref: (0, 0)),
        ],
        scratch_shapes=[
            pltpu.VMEM((8, H), jnp.float32),
            pltpu.VMEM((8, H), jnp.float32),
            pltpu.VMEM((H, _G4), jnp.float32),
            pltpu.VMEM((bt, _G4), jnp.float32),
            pltpu.VMEM((2, bt, H), jnp.float32),
            pltpu.SemaphoreType.DMA,
        ],
    )
    outputs, c_final, h_final = pl.pallas_call(
        functools.partial(_fused_kernel, block_t=bt),
        grid_spec=grid_spec,
        out_shape=(jax.ShapeDtypeStruct((s_pad, H), jnp.float32),
                   jax.ShapeDtypeStruct((1, H), jnp.float32),
                   jax.ShapeDtypeStruct((1, H), jnp.float32)),
        compiler_params=pltpu.CompilerParams(
            dimension_semantics=("arbitrary",)),
    )(tok, emb, w, b, h0_2d, c0_2d)

    if s_pad != S:
        h_final = outputs[S - 1:S]
        outputs = outputs[:S]
    outputs = outputs.reshape(S, 1, H)
    h_final = h_final.reshape(1, 1, H)
    c_final = c_final.reshape(1, 1, H)
    return outputs, (h_final, c_final)
